# Initial kernel scaffold; baseline (speedup 1.0000x reference)
import jax
import jax.numpy as jnp
from jax.experimental import pallas as pl


def kernel(x, edge_index, W_proj, b_proj, W1, b1, W2, b2, Wc, bc):
    raise NotImplementedError("write your pallas kernel here")


# speedup vs baseline: 18.5356x; 18.5356x over previous
import functools

import jax
import jax.numpy as jnp
from jax import lax
from jax.experimental import pallas as pl
from jax.experimental.pallas import tpu as pltpu
from jax.experimental.pallas import tpu_sc as plsc

N = 50000
N_MOD = 3
HID = 64
HALF = 32
E = 1600000
NC = 2
NS = 16
N_PAD = 51200
E_PAD = 1638400
ROWS = E_PAD // 128
ROWS_T = ROWS // NS
BLK_CONV = ROWS_T // 8
ROWS_W = ROWS // (NC * NS)
BLK_DEG = ROWS_W // 8
RPT = N_PAD // NS
DUMMY = N

_mesh = plsc.VectorSubcoreMesh(core_axis_name="c", subcore_axis_name="s")
_sc_params = pltpu.CompilerParams(needs_layout_passes=False,
                                  use_tc_tiling_on_sc=False)


@functools.partial(
    pl.kernel,
    out_type=jax.ShapeDtypeStruct((NC * NS, N_PAD), jnp.float32),
    mesh=_mesh,
    scratch_types=[
        pltpu.VMEM((8, 128), jnp.int32),
        pltpu.VMEM((N_PAD,), jnp.float32),
    ],
    compiler_params=_sc_params,
)
def _deg(edges, out, didx, deg):
    c = lax.axis_index("c")
    s = lax.axis_index("s")
    w = c * NS + s
    zeros = jnp.zeros((16,), jnp.float32)

    @pl.loop(0, N_PAD // 16)
    def _zero(i):
        deg[pl.ds(i * 16, 16)] = zeros

    ones = jnp.ones((16,), jnp.float32)

    @pl.loop(0, BLK_DEG)
    def _blk(b):
        row0 = (w * BLK_DEG + b) * 8
        pltpu.sync_copy(edges.at[1].at[pl.ds(row0, 8)], didx)
        for j in range(8):
            for k in range(8):
                idx = didx[j, pl.ds(k * 16, 16)]
                plsc.addupdate_scatter(deg, [idx], ones)

    pltpu.sync_copy(deg, out.at[w])


@functools.partial(
    pl.kernel,
    out_type=jax.ShapeDtypeStruct((NC, N_PAD, HALF), jnp.float32),
    mesh=_mesh,
    scratch_types=[
        pltpu.VMEM((8, 128), jnp.int32),
        pltpu.VMEM((8, 128), jnp.int32),
        pltpu.VMEM((128, HALF), jnp.float32),
        pltpu.VMEM((128, HALF), jnp.float32),
        pltpu.VMEM_SHARED((N_PAD, HALF), jnp.float32),
        pltpu.SemaphoreType.DMA,
        pltpu.SemaphoreType.DMA,
    ],
    compiler_params=_sc_params,
)
def _conv(g, edges, out, sidx, didx, bufa, bufb, acc, sga, sgb):
    c = lax.axis_index("c")
    s = lax.axis_index("s")

    @pl.loop(0, RPT // 128)
    def _init(i):
        r0 = s * RPT + i * 128
        pltpu.sync_copy(g.at[c].at[pl.ds(r0, 128)], bufa)
        pltpu.sync_copy(bufa, acc.at[pl.ds(r0, 128)])

    plsc.subcore_barrier()

    @pl.loop(0, BLK_CONV)
    def _blk(b):
        row0 = (s * BLK_CONV + b) * 8
        pltpu.sync_copy(edges.at[0].at[pl.ds(row0, 8)], sidx)
        pltpu.sync_copy(edges.at[1].at[pl.ds(row0, 8)], didx)
        for j in range(0, 8, 2):
            cpa = pltpu.async_copy(g.at[c].at[sidx.at[j]], bufa, sga)
            cpb = pltpu.async_copy(g.at[c].at[sidx.at[j + 1]], bufb, sgb)
            cpa.wait()
            pltpu.sync_copy(bufa, acc.at[didx.at[j]], add=True)
            cpb.wait()
            pltpu.sync_copy(bufb, acc.at[didx.at[j + 1]], add=True)

    plsc.subcore_barrier()

    @pl.loop(0, RPT // 128)
    def _wb(i):
        r0 = s * RPT + i * 128
        pltpu.sync_copy(acc.at[pl.ds(r0, 128)], bufa)
        pltpu.sync_copy(bufa, out.at[c].at[pl.ds(r0, 128)])


TBLK = 2048


def _tc1_body(xp_ref, degs_ref, wf_ref, bf_ref, g_ref, dinv_ref):
    deg = jnp.sum(degs_ref[...], axis=0) + 1.0
    dinv = lax.rsqrt(deg)[:, None]
    x = xp_ref[...]
    wf = wf_ref[...]
    hw = (x[:, 0:1] * wf[0:1, :] + x[:, 1:2] * wf[1:2, :]
          + x[:, 2:3] * wf[2:3, :] + bf_ref[...])
    gg = dinv * hw
    g_ref[0] = gg[:, :HALF]
    g_ref[1] = gg[:, HALF:]
    dinv_ref[...] = dinv


def _tc1(xp, degs, wf, bf):
    nb = N_PAD // TBLK
    return pl.pallas_call(
        _tc1_body,
        grid=(nb,),
        in_specs=[
            pl.BlockSpec((TBLK, N_MOD), lambda i: (i, 0)),
            pl.BlockSpec((NC * NS, TBLK), lambda i: (0, i)),
            pl.BlockSpec((N_MOD, HID), lambda i: (0, 0)),
            pl.BlockSpec((1, HID), lambda i: (0, 0)),
        ],
        out_specs=[
            pl.BlockSpec((NC, TBLK, HALF), lambda i: (0, i, 0)),
            pl.BlockSpec((TBLK, 1), lambda i: (i, 0)),
        ],
        out_shape=[
            jax.ShapeDtypeStruct((NC, N_PAD, HALF), jnp.float32),
            jax.ShapeDtypeStruct((N_PAD, 1), jnp.float32),
        ],
    )(xp, degs, wf, bf)


def _tc2_body(acc_ref, dinv_ref, w2_ref, b1_ref, g2_ref):
    accb = jnp.concatenate([acc_ref[0], acc_ref[1]], axis=1)
    dinv = dinv_ref[...]
    h1 = jnp.maximum(dinv * accb + b1_ref[...], 0.0)
    hw2 = jnp.dot(h1, w2_ref[...], preferred_element_type=jnp.float32)
    gg = dinv * hw2
    g2_ref[0] = gg[:, :HALF]
    g2_ref[1] = gg[:, HALF:]


def _tc2(acc1, dinv, w2, b1):
    nb = N_PAD // TBLK
    return pl.pallas_call(
        _tc2_body,
        grid=(nb,),
        in_specs=[
            pl.BlockSpec((NC, TBLK, HALF), lambda i: (0, i, 0)),
            pl.BlockSpec((TBLK, 1), lambda i: (i, 0)),
            pl.BlockSpec((HID, HID), lambda i: (0, 0)),
            pl.BlockSpec((1, HID), lambda i: (0, 0)),
        ],
        out_specs=pl.BlockSpec((NC, TBLK, HALF), lambda i: (0, i, 0)),
        out_shape=jax.ShapeDtypeStruct((NC, N_PAD, HALF), jnp.float32),
    )(acc1, dinv, w2, b1)


def _tc3_body(acc_ref, dinv_ref, b2_ref, wc_ref, bc_ref, out_ref, sum_ref):
    i = pl.program_id(0)

    @pl.when(i == 0)
    def _():
        sum_ref[...] = jnp.zeros_like(sum_ref)

    accb = jnp.concatenate([acc_ref[0], acc_ref[1]], axis=1)
    h2 = jnp.maximum(dinv_ref[...] * accb + b2_ref[...], 0.0)
    rows = i * TBLK + lax.broadcasted_iota(jnp.int32, (TBLK, 1), 0)
    h2 = jnp.where(rows < N, h2, 0.0)
    sum_ref[...] += jnp.sum(h2, axis=0, keepdims=True)

    @pl.when(i == pl.num_programs(0) - 1)
    def _():
        mean = sum_ref[...] * (1.0 / N)
        out_ref[...] = (jnp.dot(mean, wc_ref[...],
                                preferred_element_type=jnp.float32)
                        + bc_ref[...])


def _tc3(acc2, dinv, b2, wc, bc):
    nb = N_PAD // TBLK
    return pl.pallas_call(
        _tc3_body,
        grid=(nb,),
        in_specs=[
            pl.BlockSpec((NC, TBLK, HALF), lambda i: (0, i, 0)),
            pl.BlockSpec((TBLK, 1), lambda i: (i, 0)),
            pl.BlockSpec((1, HID), lambda i: (0, 0)),
            pl.BlockSpec((HID, 1), lambda i: (0, 0)),
            pl.BlockSpec((1, 1), lambda i: (0, 0)),
        ],
        out_specs=pl.BlockSpec((1, 1), lambda i: (0, 0)),
        out_shape=jax.ShapeDtypeStruct((1, 1), jnp.float32),
        scratch_shapes=[pltpu.VMEM((1, HID), jnp.float32)],
    )(acc2, dinv, b2, wc, bc)


@jax.jit
def _run(x, edge_index, W_proj, b_proj, W1, b1, W2, b2, Wc, bc):
    xr = x.reshape(N, N_MOD)
    xp = jnp.zeros((N_PAD, N_MOD), jnp.float32).at[:N].set(xr)
    ei = jnp.full((2, E_PAD), DUMMY, jnp.int32).at[:, :E].set(edge_index)
    edges = ei.reshape(2, ROWS, 128)
    wf = W_proj @ W1
    bf = (b_proj @ W1)[None, :]

    degs = _deg(edges)
    g1, dinv = _tc1(xp, degs, wf, bf)
    acc1 = _conv(g1, edges)
    g2 = _tc2(acc1, dinv, W2, b1[None, :])
    acc2 = _conv(g2, edges)
    return _tc3(acc2, dinv, b2[None, :], Wc, bc[None, :])


def kernel(x, edge_index, W_proj, b_proj, W1, b1, W2, b2, Wc, bc):
    return _run(x, edge_index, W_proj, b_proj, W1, b1, W2, b2, Wc, bc)


# speedup vs baseline: 24.9358x; 1.3453x over previous
import functools

import jax
import jax.numpy as jnp
from jax import lax
from jax.experimental import pallas as pl
from jax.experimental.pallas import tpu as pltpu
from jax.experimental.pallas import tpu_sc as plsc

N = 50000
N_MOD = 3
HID = 64
HALF = 32
E = 1600000
NC = 2
NS = 16
N_PAD = 51200
E_PAD = 1638400
ROWS = E_PAD // 128
ROWS_T = ROWS // NS
BLK_CONV = ROWS_T // 8
ROWS_W = ROWS // (NC * NS)
BLK_DEG = ROWS_W // 8
RPT = N_PAD // NS
DUMMY = N

_mesh = plsc.VectorSubcoreMesh(core_axis_name="c", subcore_axis_name="s")
_sc_params = pltpu.CompilerParams(needs_layout_passes=False,
                                  use_tc_tiling_on_sc=False)


@functools.partial(
    pl.kernel,
    out_type=jax.ShapeDtypeStruct((NC * NS, N_PAD), jnp.float32),
    mesh=_mesh,
    scratch_types=[
        pltpu.VMEM((8, 2, 128), jnp.int32),
        pltpu.VMEM((N_PAD,), jnp.float32),
    ],
    compiler_params=_sc_params,
)
def _deg(edges, out, didx, deg):
    c = lax.axis_index("c")
    s = lax.axis_index("s")
    w = c * NS + s
    zeros = jnp.zeros((16,), jnp.float32)

    @pl.loop(0, N_PAD // 16)
    def _zero(i):
        deg[pl.ds(i * 16, 16)] = zeros

    ones = jnp.ones((16,), jnp.float32)

    @pl.loop(0, BLK_DEG)
    def _blk(b):
        row0 = (w * BLK_DEG + b) * 8
        pltpu.sync_copy(edges.at[pl.ds(row0, 8)], didx)
        for j in range(8):
            for k in range(8):
                idx = didx[j, 1, pl.ds(k * 16, 16)]
                plsc.addupdate_scatter(deg, [idx], ones)

    pltpu.sync_copy(deg, out.at[w])


DEPTH = 5
NPAIR = ROWS_T // DEPTH


@functools.partial(
    pl.kernel,
    out_type=jax.ShapeDtypeStruct((NC, N_PAD, HALF), jnp.float32),
    mesh=_mesh,
    scratch_types=[
        pltpu.VMEM((DEPTH, 2, 128), jnp.int32),
        pltpu.VMEM((DEPTH, 2, 128), jnp.int32),
        pltpu.VMEM((DEPTH, 128, HALF), jnp.float32),
        pltpu.VMEM_SHARED((N_PAD, HALF), jnp.float32),
        pltpu.SemaphoreType.DMA,
        pltpu.SemaphoreType.DMA,
        pltpu.SemaphoreType.DMA,
    ],
    compiler_params=_sc_params,
)
def _conv(g, edges, out, ib0, ib1, bufs, acc, sem_i, sem_g, sem_s):
    c = lax.axis_index("c")
    s = lax.axis_index("s")
    base = s * ROWS_T

    @pl.loop(0, RPT // 128)
    def _init(i):
        r0 = s * RPT + i * 128
        pltpu.sync_copy(g.at[c].at[pl.ds(r0, 128)], bufs.at[0])
        pltpu.sync_copy(bufs.at[0], acc.at[pl.ds(r0, 128)])

    plsc.subcore_barrier()

    pltpu.async_copy(edges.at[pl.ds(base, DEPTH)], ib0, sem_i)

    @pl.loop(0, NPAIR // 2)
    def _pair(p):
        for off, ib, ibn in ((0, ib0, ib1), (1, ib1, ib0)):
            q = p * 2 + off
            row0 = base + q * DEPTH
            pltpu.make_async_copy(edges.at[pl.ds(row0, DEPTH)], ib,
                                  sem_i).wait()

            @pl.when(q + 1 < NPAIR)
            def _prefetch():
                pltpu.async_copy(edges.at[pl.ds(row0 + DEPTH, DEPTH)], ibn,
                                 sem_i)

            gds = [
                pltpu.async_copy(g.at[c].at[ib.at[j, 0]], bufs.at[j], sem_g)
                for j in range(DEPTH)
            ]
            sds = []
            for j in range(DEPTH):
                gds[j].wait()
                sds.append(pltpu.async_copy(bufs.at[j], acc.at[ib.at[j, 1]],
                                            sem_s, add=True))
            for d in sds:
                d.wait()

    plsc.subcore_barrier()

    @pl.loop(0, RPT // 128)
    def _wb(i):
        r0 = s * RPT + i * 128
        pltpu.sync_copy(acc.at[pl.ds(r0, 128)], bufs.at[0])
        pltpu.sync_copy(bufs.at[0], out.at[c].at[pl.ds(r0, 128)])


TBLK = 2048


def _tc1_body(xp_ref, degs_ref, wf_ref, bf_ref, g_ref, dinv_ref):
    deg = jnp.sum(degs_ref[...], axis=0) + 1.0
    dinv = lax.rsqrt(deg)[:, None]
    x = xp_ref[...]
    wf = wf_ref[...]
    hw = (x[:, 0:1] * wf[0:1, :] + x[:, 1:2] * wf[1:2, :]
          + x[:, 2:3] * wf[2:3, :] + bf_ref[...])
    gg = dinv * hw
    g_ref[0] = gg[:, :HALF]
    g_ref[1] = gg[:, HALF:]
    dinv_ref[...] = dinv


def _tc1(xp, degs, wf, bf):
    nb = N_PAD // TBLK
    return pl.pallas_call(
        _tc1_body,
        grid=(nb,),
        in_specs=[
            pl.BlockSpec((TBLK, N_MOD), lambda i: (i, 0)),
            pl.BlockSpec((NC * NS, TBLK), lambda i: (0, i)),
            pl.BlockSpec((N_MOD, HID), lambda i: (0, 0)),
            pl.BlockSpec((1, HID), lambda i: (0, 0)),
        ],
        out_specs=[
            pl.BlockSpec((NC, TBLK, HALF), lambda i: (0, i, 0)),
            pl.BlockSpec((TBLK, 1), lambda i: (i, 0)),
        ],
        out_shape=[
            jax.ShapeDtypeStruct((NC, N_PAD, HALF), jnp.float32),
            jax.ShapeDtypeStruct((N_PAD, 1), jnp.float32),
        ],
    )(xp, degs, wf, bf)


def _tc2_body(acc_ref, dinv_ref, w2_ref, b1_ref, g2_ref):
    accb = jnp.concatenate([acc_ref[0], acc_ref[1]], axis=1)
    dinv = dinv_ref[...]
    h1 = jnp.maximum(dinv * accb + b1_ref[...], 0.0)
    hw2 = jnp.dot(h1, w2_ref[...], preferred_element_type=jnp.float32)
    gg = dinv * hw2
    g2_ref[0] = gg[:, :HALF]
    g2_ref[1] = gg[:, HALF:]


def _tc2(acc1, dinv, w2, b1):
    nb = N_PAD // TBLK
    return pl.pallas_call(
        _tc2_body,
        grid=(nb,),
        in_specs=[
            pl.BlockSpec((NC, TBLK, HALF), lambda i: (0, i, 0)),
            pl.BlockSpec((TBLK, 1), lambda i: (i, 0)),
            pl.BlockSpec((HID, HID), lambda i: (0, 0)),
            pl.BlockSpec((1, HID), lambda i: (0, 0)),
        ],
        out_specs=pl.BlockSpec((NC, TBLK, HALF), lambda i: (0, i, 0)),
        out_shape=jax.ShapeDtypeStruct((NC, N_PAD, HALF), jnp.float32),
    )(acc1, dinv, w2, b1)


def _tc3_body(acc_ref, dinv_ref, b2_ref, wc_ref, bc_ref, out_ref, sum_ref):
    i = pl.program_id(0)

    @pl.when(i == 0)
    def _():
        sum_ref[...] = jnp.zeros_like(sum_ref)

    accb = jnp.concatenate([acc_ref[0], acc_ref[1]], axis=1)
    h2 = jnp.maximum(dinv_ref[...] * accb + b2_ref[...], 0.0)
    rows = i * TBLK + lax.broadcasted_iota(jnp.int32, (TBLK, 1), 0)
    h2 = jnp.where(rows < N, h2, 0.0)
    sum_ref[...] += jnp.sum(h2, axis=0, keepdims=True)

    @pl.when(i == pl.num_programs(0) - 1)
    def _():
        mean = sum_ref[...] * (1.0 / N)
        out_ref[...] = (jnp.dot(mean, wc_ref[...],
                                preferred_element_type=jnp.float32)
                        + bc_ref[...])


def _tc3(acc2, dinv, b2, wc, bc):
    nb = N_PAD // TBLK
    return pl.pallas_call(
        _tc3_body,
        grid=(nb,),
        in_specs=[
            pl.BlockSpec((NC, TBLK, HALF), lambda i: (0, i, 0)),
            pl.BlockSpec((TBLK, 1), lambda i: (i, 0)),
            pl.BlockSpec((1, HID), lambda i: (0, 0)),
            pl.BlockSpec((HID, 1), lambda i: (0, 0)),
            pl.BlockSpec((1, 1), lambda i: (0, 0)),
        ],
        out_specs=pl.BlockSpec((1, 1), lambda i: (0, 0)),
        out_shape=jax.ShapeDtypeStruct((1, 1), jnp.float32),
        scratch_shapes=[pltpu.VMEM((1, HID), jnp.float32)],
    )(acc2, dinv, b2, wc, bc)


@jax.jit
def _run(x, edge_index, W_proj, b_proj, W1, b1, W2, b2, Wc, bc):
    xr = x.reshape(N, N_MOD)
    xp = jnp.zeros((N_PAD, N_MOD), jnp.float32).at[:N].set(xr)
    ei = jnp.full((2, E_PAD), DUMMY, jnp.int32).at[:, :E].set(edge_index)
    edges = ei.reshape(2, ROWS, 128).transpose(1, 0, 2)
    wf = W_proj @ W1
    bf = (b_proj @ W1)[None, :]

    degs = _deg(edges)
    g1, dinv = _tc1(xp, degs, wf, bf)
    acc1 = _conv(g1, edges)
    g2 = _tc2(acc1, dinv, W2, b1[None, :])
    acc2 = _conv(g2, edges)
    return _tc3(acc2, dinv, b2[None, :], Wc, bc[None, :])


def kernel(x, edge_index, W_proj, b_proj, W1, b1, W2, b2, Wc, bc):
    return _run(x, edge_index, W_proj, b_proj, W1, b1, W2, b2, Wc, bc)


# speedup vs baseline: 25.8190x; 1.0354x over previous
import functools

import jax
import jax.numpy as jnp
from jax import lax
from jax.experimental import pallas as pl
from jax.experimental.pallas import tpu as pltpu
from jax.experimental.pallas import tpu_sc as plsc

N = 50000
N_MOD = 3
HID = 64
HALF = 32
E = 1600000
NC = 2
NS = 16
N_PAD = 51200
E_PAD = 1638400
ROWS = E_PAD // 128
ROWS_T = ROWS // NS
BLK_CONV = ROWS_T // 8
ROWS_W = ROWS // (NC * NS)
BLK_DEG = ROWS_W // 8
RPT = N_PAD // NS
DUMMY = N

_mesh = plsc.VectorSubcoreMesh(core_axis_name="c", subcore_axis_name="s")
_sc_params = pltpu.CompilerParams(needs_layout_passes=False,
                                  use_tc_tiling_on_sc=False)


@functools.partial(
    pl.kernel,
    out_type=jax.ShapeDtypeStruct((NC * NS, N_PAD), jnp.float32),
    mesh=_mesh,
    scratch_types=[
        pltpu.VMEM((8, 2, 128), jnp.int32),
        pltpu.VMEM((N_PAD,), jnp.float32),
    ],
    compiler_params=_sc_params,
)
def _deg(edges, out, didx, deg):
    c = lax.axis_index("c")
    s = lax.axis_index("s")
    w = c * NS + s
    zeros = jnp.zeros((16,), jnp.float32)

    @pl.loop(0, N_PAD // 16)
    def _zero(i):
        deg[pl.ds(i * 16, 16)] = zeros

    ones = jnp.ones((16,), jnp.float32)

    @pl.loop(0, BLK_DEG)
    def _blk(b):
        row0 = (w * BLK_DEG + b) * 8
        pltpu.sync_copy(edges.at[pl.ds(row0, 8)], didx)
        for j in range(8):
            for k in range(8):
                idx = didx[j, 1, pl.ds(k * 16, 16)]
                plsc.addupdate_scatter(deg, [idx], ones)

    pltpu.sync_copy(deg, out.at[w])


DEPTH = 5
NPAIR = ROWS_T // DEPTH


@functools.partial(
    pl.kernel,
    out_type=jax.ShapeDtypeStruct((NC, N_PAD, HALF), jnp.float32),
    mesh=_mesh,
    scratch_types=[
        pltpu.VMEM((DEPTH, 2, 128), jnp.int32),
        pltpu.VMEM((DEPTH, 2, 128), jnp.int32),
        pltpu.VMEM((DEPTH, 128), jnp.int32),
        pltpu.VMEM((DEPTH, 128, HALF), jnp.float32),
        pltpu.VMEM_SHARED((N_PAD, HALF), jnp.float32),
        pltpu.SemaphoreType.DMA,
        pltpu.SemaphoreType.DMA,
        pltpu.SemaphoreType.DMA,
    ],
    compiler_params=_sc_params,
)
def _conv(g, edges, out, ib0, ib1, dbuf, bufs, acc, sem_i, sem_g, sem_s):
    c = lax.axis_index("c")
    s = lax.axis_index("s")
    base = s * ROWS_T

    @pl.loop(0, RPT // 128)
    def _init(i):
        r0 = s * RPT + i * 128
        pltpu.sync_copy(g.at[c].at[pl.ds(r0, 128)], bufs.at[0])
        pltpu.sync_copy(bufs.at[0], acc.at[pl.ds(r0, 128)])

    plsc.subcore_barrier()

    pltpu.async_copy(edges.at[pl.ds(base, DEPTH)], ib0, sem_i)

    @pl.loop(0, NPAIR // 2)
    def _pair(p):
        for off, ib, ibn in ((0, ib0, ib1), (1, ib1, ib0)):
            q = p * 2 + off
            row0 = base + q * DEPTH
            pltpu.make_async_copy(edges.at[pl.ds(row0, DEPTH)], ib,
                                  sem_i).wait()

            @pl.when(q + 1 < NPAIR)
            def _prefetch():
                pltpu.async_copy(edges.at[pl.ds(row0 + DEPTH, DEPTH)], ibn,
                                 sem_i)

            for j in range(DEPTH):
                @pl.when(q > 0)
                def _wait_prev_scatter():
                    pltpu.make_async_copy(bufs.at[j], acc.at[dbuf.at[j]],
                                          sem_s).wait()
                pltpu.async_copy(g.at[c].at[ib.at[j, 0]], bufs.at[j], sem_g)
            for j in range(DEPTH):
                pltpu.make_async_copy(g.at[c].at[ib.at[j, 0]], bufs.at[j],
                                      sem_g).wait()
                for k in range(8):
                    dbuf[j, pl.ds(k * 16, 16)] = ib[j, 1, pl.ds(k * 16, 16)]
                pltpu.async_copy(bufs.at[j], acc.at[dbuf.at[j]], sem_s,
                                 add=True)

    for j in range(DEPTH):
        pltpu.make_async_copy(bufs.at[j], acc.at[dbuf.at[j]], sem_s).wait()

    plsc.subcore_barrier()

    @pl.loop(0, RPT // 128)
    def _wb(i):
        r0 = s * RPT + i * 128
        pltpu.sync_copy(acc.at[pl.ds(r0, 128)], bufs.at[0])
        pltpu.sync_copy(bufs.at[0], out.at[c].at[pl.ds(r0, 128)])


TBLK = 2048


def _tc1_body(xp_ref, degs_ref, wf_ref, bf_ref, g_ref, dinv_ref):
    deg = jnp.sum(degs_ref[...], axis=0) + 1.0
    dinv = lax.rsqrt(deg)[:, None]
    x = xp_ref[...]
    wf = wf_ref[...]
    hw = (x[:, 0:1] * wf[0:1, :] + x[:, 1:2] * wf[1:2, :]
          + x[:, 2:3] * wf[2:3, :] + bf_ref[...])
    gg = dinv * hw
    g_ref[0] = gg[:, :HALF]
    g_ref[1] = gg[:, HALF:]
    dinv_ref[...] = dinv


def _tc1(xp, degs, wf, bf):
    nb = N_PAD // TBLK
    return pl.pallas_call(
        _tc1_body,
        grid=(nb,),
        in_specs=[
            pl.BlockSpec((TBLK, N_MOD), lambda i: (i, 0)),
            pl.BlockSpec((NC * NS, TBLK), lambda i: (0, i)),
            pl.BlockSpec((N_MOD, HID), lambda i: (0, 0)),
            pl.BlockSpec((1, HID), lambda i: (0, 0)),
        ],
        out_specs=[
            pl.BlockSpec((NC, TBLK, HALF), lambda i: (0, i, 0)),
            pl.BlockSpec((TBLK, 1), lambda i: (i, 0)),
        ],
        out_shape=[
            jax.ShapeDtypeStruct((NC, N_PAD, HALF), jnp.float32),
            jax.ShapeDtypeStruct((N_PAD, 1), jnp.float32),
        ],
    )(xp, degs, wf, bf)


def _tc2_body(acc_ref, dinv_ref, w2_ref, b1_ref, g2_ref):
    accb = jnp.concatenate([acc_ref[0], acc_ref[1]], axis=1)
    dinv = dinv_ref[...]
    h1 = jnp.maximum(dinv * accb + b1_ref[...], 0.0)
    hw2 = jnp.dot(h1, w2_ref[...], preferred_element_type=jnp.float32)
    gg = dinv * hw2
    g2_ref[0] = gg[:, :HALF]
    g2_ref[1] = gg[:, HALF:]


def _tc2(acc1, dinv, w2, b1):
    nb = N_PAD // TBLK
    return pl.pallas_call(
        _tc2_body,
        grid=(nb,),
        in_specs=[
            pl.BlockSpec((NC, TBLK, HALF), lambda i: (0, i, 0)),
            pl.BlockSpec((TBLK, 1), lambda i: (i, 0)),
            pl.BlockSpec((HID, HID), lambda i: (0, 0)),
            pl.BlockSpec((1, HID), lambda i: (0, 0)),
        ],
        out_specs=pl.BlockSpec((NC, TBLK, HALF), lambda i: (0, i, 0)),
        out_shape=jax.ShapeDtypeStruct((NC, N_PAD, HALF), jnp.float32),
    )(acc1, dinv, w2, b1)


def _tc3_body(acc_ref, dinv_ref, b2_ref, wc_ref, bc_ref, out_ref, sum_ref):
    i = pl.program_id(0)

    @pl.when(i == 0)
    def _():
        sum_ref[...] = jnp.zeros_like(sum_ref)

    accb = jnp.concatenate([acc_ref[0], acc_ref[1]], axis=1)
    h2 = jnp.maximum(dinv_ref[...] * accb + b2_ref[...], 0.0)
    rows = i * TBLK + lax.broadcasted_iota(jnp.int32, (TBLK, 1), 0)
    h2 = jnp.where(rows < N, h2, 0.0)
    sum_ref[...] += jnp.sum(h2, axis=0, keepdims=True)

    @pl.when(i == pl.num_programs(0) - 1)
    def _():
        mean = sum_ref[...] * (1.0 / N)
        out_ref[...] = (jnp.dot(mean, wc_ref[...],
                                preferred_element_type=jnp.float32)
                        + bc_ref[...])


def _tc3(acc2, dinv, b2, wc, bc):
    nb = N_PAD // TBLK
    return pl.pallas_call(
        _tc3_body,
        grid=(nb,),
        in_specs=[
            pl.BlockSpec((NC, TBLK, HALF), lambda i: (0, i, 0)),
            pl.BlockSpec((TBLK, 1), lambda i: (i, 0)),
            pl.BlockSpec((1, HID), lambda i: (0, 0)),
            pl.BlockSpec((HID, 1), lambda i: (0, 0)),
            pl.BlockSpec((1, 1), lambda i: (0, 0)),
        ],
        out_specs=pl.BlockSpec((1, 1), lambda i: (0, 0)),
        out_shape=jax.ShapeDtypeStruct((1, 1), jnp.float32),
        scratch_shapes=[pltpu.VMEM((1, HID), jnp.float32)],
    )(acc2, dinv, b2, wc, bc)


@jax.jit
def _run(x, edge_index, W_proj, b_proj, W1, b1, W2, b2, Wc, bc):
    xr = x.reshape(N, N_MOD)
    xp = jnp.zeros((N_PAD, N_MOD), jnp.float32).at[:N].set(xr)
    ei = jnp.full((2, E_PAD), DUMMY, jnp.int32).at[:, :E].set(edge_index)
    edges = ei.reshape(2, ROWS, 128).transpose(1, 0, 2)
    wf = W_proj @ W1
    bf = (b_proj @ W1)[None, :]

    degs = _deg(edges)
    g1, dinv = _tc1(xp, degs, wf, bf)
    acc1 = _conv(g1, edges)
    g2 = _tc2(acc1, dinv, W2, b1[None, :])
    acc2 = _conv(g2, edges)
    return _tc3(acc2, dinv, b2[None, :], Wc, bc[None, :])


def kernel(x, edge_index, W_proj, b_proj, W1, b1, W2, b2, Wc, bc):
    return _run(x, edge_index, W_proj, b_proj, W1, b1, W2, b2, Wc, bc)


# speedup vs baseline: 35.7716x; 1.3855x over previous
import functools

import jax
import jax.numpy as jnp
from jax import lax
from jax.experimental import pallas as pl
from jax.experimental.pallas import tpu as pltpu
from jax.experimental.pallas import tpu_sc as plsc

N = 50000
N_MOD = 3
HID = 64
HALF = 32
E = 1600000
NC = 2
NS = 16
N_PAD = 51200
E_PAD = 1638400
ROWS = E_PAD // 128
ROWS_T = ROWS // NS
BLK_CONV = ROWS_T // 8
ROWS_W = ROWS // (NC * NS)
BLK_DEG = ROWS_W // 8
RPT = N_PAD // NS
DUMMY = N

_mesh = plsc.VectorSubcoreMesh(core_axis_name="c", subcore_axis_name="s")
_sc_params = pltpu.CompilerParams(needs_layout_passes=False,
                                  use_tc_tiling_on_sc=False)


@functools.partial(
    pl.kernel,
    out_type=jax.ShapeDtypeStruct((NC * NS, N_PAD), jnp.float32),
    mesh=_mesh,
    scratch_types=[
        pltpu.VMEM((8, 2, 128), jnp.int32),
        pltpu.VMEM((N_PAD,), jnp.float32),
    ],
    compiler_params=_sc_params,
)
def _deg(edges, out, didx, deg):
    c = lax.axis_index("c")
    s = lax.axis_index("s")
    w = c * NS + s
    zeros = jnp.zeros((16,), jnp.float32)

    @pl.loop(0, N_PAD // 16)
    def _zero(i):
        deg[pl.ds(i * 16, 16)] = zeros

    ones = jnp.ones((16,), jnp.float32)

    @pl.loop(0, BLK_DEG)
    def _blk(b):
        row0 = (w * BLK_DEG + b) * 8
        pltpu.sync_copy(edges.at[pl.ds(row0, 8)], didx)
        for j in range(8):
            for k in range(8):
                idx = didx[j, 1, pl.ds(k * 16, 16)]
                plsc.addupdate_scatter(deg, [idx], ones)

    pltpu.sync_copy(deg, out.at[w])


DEPTH = 16
NPAIR = ROWS_T // DEPTH


@functools.partial(
    pl.kernel,
    out_type=jax.ShapeDtypeStruct((NC, N_PAD, HALF), jnp.bfloat16),
    mesh=_mesh,
    scratch_types=[
        pltpu.VMEM((DEPTH, 2, 128), jnp.int32),
        pltpu.VMEM((DEPTH, 2, 128), jnp.int32),
        pltpu.VMEM((DEPTH, 128), jnp.int32),
        pltpu.VMEM((DEPTH, 128, HALF), jnp.bfloat16),
        pltpu.VMEM_SHARED((N_PAD, HALF), jnp.bfloat16),
        pltpu.SemaphoreType.DMA,
        pltpu.SemaphoreType.DMA,
        pltpu.SemaphoreType.DMA,
    ],
    compiler_params=_sc_params,
)
def _conv(g, edges, out, ib0, ib1, dbuf, bufs, acc, sem_i, sem_g, sem_s):
    c = lax.axis_index("c")
    s = lax.axis_index("s")
    base = s * ROWS_T

    @pl.loop(0, RPT // 128)
    def _init(i):
        r0 = s * RPT + i * 128
        pltpu.sync_copy(g.at[c].at[pl.ds(r0, 128)], bufs.at[0])
        pltpu.sync_copy(bufs.at[0], acc.at[pl.ds(r0, 128)])

    plsc.subcore_barrier()

    pltpu.async_copy(edges.at[pl.ds(base, DEPTH)], ib0, sem_i)

    @pl.loop(0, NPAIR // 2)
    def _pair(p):
        for off, ib, ibn in ((0, ib0, ib1), (1, ib1, ib0)):
            q = p * 2 + off
            row0 = base + q * DEPTH
            pltpu.make_async_copy(edges.at[pl.ds(row0, DEPTH)], ib,
                                  sem_i).wait()

            @pl.when(q + 1 < NPAIR)
            def _prefetch():
                pltpu.async_copy(edges.at[pl.ds(row0 + DEPTH, DEPTH)], ibn,
                                 sem_i)

            for j in range(DEPTH):
                @pl.when(q > 0)
                def _wait_prev_scatter():
                    pltpu.make_async_copy(bufs.at[j], acc.at[dbuf.at[j]],
                                          sem_s).wait()
                pltpu.async_copy(g.at[c].at[ib.at[j, 0]], bufs.at[j], sem_g)
            for j in range(DEPTH):
                pltpu.make_async_copy(g.at[c].at[ib.at[j, 0]], bufs.at[j],
                                      sem_g).wait()
                for k in range(8):
                    dbuf[j, pl.ds(k * 16, 16)] = ib[j, 1, pl.ds(k * 16, 16)]
                pltpu.async_copy(bufs.at[j], acc.at[dbuf.at[j]], sem_s,
                                 add=True)

    for j in range(DEPTH):
        pltpu.make_async_copy(bufs.at[j], acc.at[dbuf.at[j]], sem_s).wait()

    plsc.subcore_barrier()

    @pl.loop(0, RPT // 128)
    def _wb(i):
        r0 = s * RPT + i * 128
        pltpu.sync_copy(acc.at[pl.ds(r0, 128)], bufs.at[0])
        pltpu.sync_copy(bufs.at[0], out.at[c].at[pl.ds(r0, 128)])


TBLK = 2048


def _tc1_body(xp_ref, degs_ref, wf_ref, bf_ref, g_ref, dinv_ref):
    deg = jnp.sum(degs_ref[...], axis=0) + 1.0
    dinv = lax.rsqrt(deg)[:, None]
    x = xp_ref[...]
    wf = wf_ref[...]
    hw = (x[:, 0:1] * wf[0:1, :] + x[:, 1:2] * wf[1:2, :]
          + x[:, 2:3] * wf[2:3, :] + bf_ref[...])
    gg = (dinv * hw).astype(jnp.bfloat16)
    g_ref[0] = gg[:, :HALF]
    g_ref[1] = gg[:, HALF:]
    dinv_ref[...] = dinv


def _tc1(xp, degs, wf, bf):
    nb = N_PAD // TBLK
    return pl.pallas_call(
        _tc1_body,
        grid=(nb,),
        in_specs=[
            pl.BlockSpec((TBLK, N_MOD), lambda i: (i, 0)),
            pl.BlockSpec((NC * NS, TBLK), lambda i: (0, i)),
            pl.BlockSpec((N_MOD, HID), lambda i: (0, 0)),
            pl.BlockSpec((1, HID), lambda i: (0, 0)),
        ],
        out_specs=[
            pl.BlockSpec((NC, TBLK, HALF), lambda i: (0, i, 0)),
            pl.BlockSpec((TBLK, 1), lambda i: (i, 0)),
        ],
        out_shape=[
            jax.ShapeDtypeStruct((NC, N_PAD, HALF), jnp.bfloat16),
            jax.ShapeDtypeStruct((N_PAD, 1), jnp.float32),
        ],
    )(xp, degs, wf, bf)


def _tc2_body(acc_ref, dinv_ref, w2_ref, b1_ref, g2_ref):
    accb = jnp.concatenate([acc_ref[0], acc_ref[1]],
                           axis=1).astype(jnp.float32)
    dinv = dinv_ref[...]
    h1 = jnp.maximum(dinv * accb + b1_ref[...], 0.0)
    hw2 = jnp.dot(h1, w2_ref[...], preferred_element_type=jnp.float32)
    gg = (dinv * hw2).astype(jnp.bfloat16)
    g2_ref[0] = gg[:, :HALF]
    g2_ref[1] = gg[:, HALF:]


def _tc2(acc1, dinv, w2, b1):
    nb = N_PAD // TBLK
    return pl.pallas_call(
        _tc2_body,
        grid=(nb,),
        in_specs=[
            pl.BlockSpec((NC, TBLK, HALF), lambda i: (0, i, 0)),
            pl.BlockSpec((TBLK, 1), lambda i: (i, 0)),
            pl.BlockSpec((HID, HID), lambda i: (0, 0)),
            pl.BlockSpec((1, HID), lambda i: (0, 0)),
        ],
        out_specs=pl.BlockSpec((NC, TBLK, HALF), lambda i: (0, i, 0)),
        out_shape=jax.ShapeDtypeStruct((NC, N_PAD, HALF), jnp.bfloat16),
    )(acc1, dinv, w2, b1)


def _tc3_body(acc_ref, dinv_ref, b2_ref, wc_ref, bc_ref, out_ref, sum_ref):
    i = pl.program_id(0)

    @pl.when(i == 0)
    def _():
        sum_ref[...] = jnp.zeros_like(sum_ref)

    accb = jnp.concatenate([acc_ref[0], acc_ref[1]],
                           axis=1).astype(jnp.float32)
    h2 = jnp.maximum(dinv_ref[...] * accb + b2_ref[...], 0.0)
    rows = i * TBLK + lax.broadcasted_iota(jnp.int32, (TBLK, 1), 0)
    h2 = jnp.where(rows < N, h2, 0.0)
    sum_ref[...] += jnp.sum(h2, axis=0, keepdims=True)

    @pl.when(i == pl.num_programs(0) - 1)
    def _():
        mean = sum_ref[...] * (1.0 / N)
        out_ref[...] = (jnp.dot(mean, wc_ref[...],
                                preferred_element_type=jnp.float32)
                        + bc_ref[...])


def _tc3(acc2, dinv, b2, wc, bc):
    nb = N_PAD // TBLK
    return pl.pallas_call(
        _tc3_body,
        grid=(nb,),
        in_specs=[
            pl.BlockSpec((NC, TBLK, HALF), lambda i: (0, i, 0)),
            pl.BlockSpec((TBLK, 1), lambda i: (i, 0)),
            pl.BlockSpec((1, HID), lambda i: (0, 0)),
            pl.BlockSpec((HID, 1), lambda i: (0, 0)),
            pl.BlockSpec((1, 1), lambda i: (0, 0)),
        ],
        out_specs=pl.BlockSpec((1, 1), lambda i: (0, 0)),
        out_shape=jax.ShapeDtypeStruct((1, 1), jnp.float32),
        scratch_shapes=[pltpu.VMEM((1, HID), jnp.float32)],
    )(acc2, dinv, b2, wc, bc)


@jax.jit
def _run(x, edge_index, W_proj, b_proj, W1, b1, W2, b2, Wc, bc):
    xr = x.reshape(N, N_MOD)
    xp = jnp.zeros((N_PAD, N_MOD), jnp.float32).at[:N].set(xr)
    ei = jnp.full((2, E_PAD), DUMMY, jnp.int32).at[:, :E].set(edge_index)
    edges = ei.reshape(2, ROWS, 128).transpose(1, 0, 2)
    wf = W_proj @ W1
    bf = (b_proj @ W1)[None, :]

    degs = _deg(edges)
    g1, dinv = _tc1(xp, degs, wf, bf)
    acc1 = _conv(g1, edges)
    g2 = _tc2(acc1, dinv, W2, b1[None, :])
    acc2 = _conv(g2, edges)
    return _tc3(acc2, dinv, b2[None, :], Wc, bc[None, :])


def kernel(x, edge_index, W_proj, b_proj, W1, b1, W2, b2, Wc, bc):
    return _run(x, edge_index, W_proj, b_proj, W1, b1, W2, b2, Wc, bc)


# speedup vs baseline: 58.1228x; 1.6248x over previous
import functools

import jax
import jax.numpy as jnp
from jax import lax
from jax.experimental import pallas as pl
from jax.experimental.pallas import tpu as pltpu
from jax.experimental.pallas import tpu_sc as plsc

N = 50000
N_MOD = 3
HID = 64
HALF = 32
E = 1600000
NC = 2
NS = 16
N_PAD = 51200
E_PAD = 1638400
ROWS = E_PAD // 128
ROWS_T = ROWS // NS
BLK_CONV = ROWS_T // 8
ROWS_W = ROWS // (NC * NS)
BLK_DEG = ROWS_W // 8
RPT = N_PAD // NS
DUMMY = N

_mesh = plsc.VectorSubcoreMesh(core_axis_name="c", subcore_axis_name="s")
_sc_params = pltpu.CompilerParams(needs_layout_passes=False,
                                  use_tc_tiling_on_sc=False)


@functools.partial(
    pl.kernel,
    out_type=jax.ShapeDtypeStruct((NC * NS, N_PAD), jnp.float32),
    mesh=_mesh,
    scratch_types=[
        pltpu.VMEM((8, 2, 128), jnp.int32),
        pltpu.VMEM((8, 2, 128), jnp.int32),
        pltpu.VMEM((N_PAD,), jnp.float32),
        pltpu.SemaphoreType.DMA,
    ],
    compiler_params=_sc_params,
)
def _deg(edges, out, didx0, didx1, deg, sem_i):
    c = lax.axis_index("c")
    s = lax.axis_index("s")
    w = c * NS + s
    zeros = jnp.zeros((16,), jnp.float32)

    @pl.loop(0, N_PAD // 16)
    def _zero(i):
        deg[pl.ds(i * 16, 16)] = zeros

    ones = jnp.ones((16,), jnp.float32)
    base = w * BLK_DEG * 8
    pltpu.async_copy(edges.at[pl.ds(base, 8)], didx0, sem_i)

    @pl.loop(0, BLK_DEG // 2)
    def _blk(b):
        for off, db, dbn in ((0, didx0, didx1), (1, didx1, didx0)):
            q = b * 2 + off
            row0 = base + q * 8
            pltpu.make_async_copy(edges.at[pl.ds(row0, 8)], db, sem_i).wait()

            @pl.when(q + 1 < BLK_DEG)
            def _prefetch():
                pltpu.async_copy(edges.at[pl.ds(row0 + 8, 8)], dbn, sem_i)

            for j in range(8):
                for k in range(8):
                    idx = db[j, 1, pl.ds(k * 16, 16)]
                    plsc.addupdate_scatter(deg, [idx], ones)

    pltpu.sync_copy(deg, out.at[w])


DEPTH = 8
NPAIR = ROWS_T // DEPTH


@functools.partial(
    pl.kernel,
    out_type=jax.ShapeDtypeStruct((NC, N_PAD, HALF), jnp.bfloat16),
    mesh=_mesh,
    scratch_types=[
        pltpu.VMEM((DEPTH, 2, 128), jnp.int32),
        pltpu.VMEM((DEPTH, 2, 128), jnp.int32),
        pltpu.VMEM((DEPTH, 128), jnp.int32),
        pltpu.VMEM((DEPTH, 128, HALF), jnp.bfloat16),
        pltpu.VMEM_SHARED((N_PAD, HALF), jnp.bfloat16),
        pltpu.VMEM_SHARED((N_PAD, HALF), jnp.bfloat16),
        pltpu.SemaphoreType.DMA,
        pltpu.SemaphoreType.DMA,
        pltpu.SemaphoreType.DMA,
    ],
    compiler_params=_sc_params,
)
def _conv(g, edges, out, ib0, ib1, dbuf, bufs, acc, gtab, sem_i, sem_g,
          sem_s):
    c = lax.axis_index("c")
    s = lax.axis_index("s")
    base = s * ROWS_T

    @pl.loop(0, RPT // 128)
    def _init(i):
        r0 = s * RPT + i * 128
        pltpu.sync_copy(g.at[c].at[pl.ds(r0, 128)], bufs.at[0])
        pltpu.sync_copy(bufs.at[0], acc.at[pl.ds(r0, 128)])
        pltpu.sync_copy(bufs.at[0], gtab.at[pl.ds(r0, 128)])

    plsc.subcore_barrier()

    pltpu.async_copy(edges.at[pl.ds(base, DEPTH)], ib0, sem_i)

    @pl.loop(0, NPAIR // 2)
    def _pair(p):
        for off, ib, ibn in ((0, ib0, ib1), (1, ib1, ib0)):
            q = p * 2 + off
            row0 = base + q * DEPTH
            pltpu.make_async_copy(edges.at[pl.ds(row0, DEPTH)], ib,
                                  sem_i).wait()

            @pl.when(q + 1 < NPAIR)
            def _prefetch():
                pltpu.async_copy(edges.at[pl.ds(row0 + DEPTH, DEPTH)], ibn,
                                 sem_i)

            for j in range(DEPTH):
                @pl.when(q > 0)
                def _wait_prev_scatter():
                    pltpu.make_async_copy(bufs.at[j], acc.at[dbuf.at[j]],
                                          sem_s).wait()
                pltpu.async_copy(gtab.at[ib.at[j, 0]], bufs.at[j], sem_g)
            for j in range(DEPTH):
                pltpu.make_async_copy(gtab.at[ib.at[j, 0]], bufs.at[j],
                                      sem_g).wait()
                for k in range(8):
                    dbuf[j, pl.ds(k * 16, 16)] = ib[j, 1, pl.ds(k * 16, 16)]
                pltpu.async_copy(bufs.at[j], acc.at[dbuf.at[j]], sem_s,
                                 add=True)

    for j in range(DEPTH):
        pltpu.make_async_copy(bufs.at[j], acc.at[dbuf.at[j]], sem_s).wait()

    plsc.subcore_barrier()

    @pl.loop(0, RPT // 128)
    def _wb(i):
        r0 = s * RPT + i * 128
        pltpu.sync_copy(acc.at[pl.ds(r0, 128)], bufs.at[0])
        pltpu.sync_copy(bufs.at[0], out.at[c].at[pl.ds(r0, 128)])


TBLK = 2048


def _tc1_body(xp_ref, degs_ref, wf_ref, bf_ref, g_ref, dinv_ref):
    deg = jnp.sum(degs_ref[...], axis=0) + 1.0
    dinv = lax.rsqrt(deg)[:, None]
    x = xp_ref[...]
    wf = wf_ref[...]
    hw = (x[:, 0:1] * wf[0:1, :] + x[:, 1:2] * wf[1:2, :]
          + x[:, 2:3] * wf[2:3, :] + bf_ref[...])
    gg = (dinv * hw).astype(jnp.bfloat16)
    g_ref[0] = gg[:, :HALF]
    g_ref[1] = gg[:, HALF:]
    dinv_ref[...] = dinv


def _tc1(xp, degs, wf, bf):
    nb = N_PAD // TBLK
    return pl.pallas_call(
        _tc1_body,
        grid=(nb,),
        in_specs=[
            pl.BlockSpec((TBLK, N_MOD), lambda i: (i, 0)),
            pl.BlockSpec((NC * NS, TBLK), lambda i: (0, i)),
            pl.BlockSpec((N_MOD, HID), lambda i: (0, 0)),
            pl.BlockSpec((1, HID), lambda i: (0, 0)),
        ],
        out_specs=[
            pl.BlockSpec((NC, TBLK, HALF), lambda i: (0, i, 0)),
            pl.BlockSpec((TBLK, 1), lambda i: (i, 0)),
        ],
        out_shape=[
            jax.ShapeDtypeStruct((NC, N_PAD, HALF), jnp.bfloat16),
            jax.ShapeDtypeStruct((N_PAD, 1), jnp.float32),
        ],
    )(xp, degs, wf, bf)


def _tc2_body(acc_ref, dinv_ref, w2_ref, b1_ref, g2_ref):
    accb = jnp.concatenate([acc_ref[0], acc_ref[1]],
                           axis=1).astype(jnp.float32)
    dinv = dinv_ref[...]
    h1 = jnp.maximum(dinv * accb + b1_ref[...], 0.0)
    hw2 = jnp.dot(h1, w2_ref[...], preferred_element_type=jnp.float32)
    gg = (dinv * hw2).astype(jnp.bfloat16)
    g2_ref[0] = gg[:, :HALF]
    g2_ref[1] = gg[:, HALF:]


def _tc2(acc1, dinv, w2, b1):
    nb = N_PAD // TBLK
    return pl.pallas_call(
        _tc2_body,
        grid=(nb,),
        in_specs=[
            pl.BlockSpec((NC, TBLK, HALF), lambda i: (0, i, 0)),
            pl.BlockSpec((TBLK, 1), lambda i: (i, 0)),
            pl.BlockSpec((HID, HID), lambda i: (0, 0)),
            pl.BlockSpec((1, HID), lambda i: (0, 0)),
        ],
        out_specs=pl.BlockSpec((NC, TBLK, HALF), lambda i: (0, i, 0)),
        out_shape=jax.ShapeDtypeStruct((NC, N_PAD, HALF), jnp.bfloat16),
    )(acc1, dinv, w2, b1)


def _tc3_body(acc_ref, dinv_ref, b2_ref, wc_ref, bc_ref, out_ref, sum_ref):
    i = pl.program_id(0)

    @pl.when(i == 0)
    def _():
        sum_ref[...] = jnp.zeros_like(sum_ref)

    accb = jnp.concatenate([acc_ref[0], acc_ref[1]],
                           axis=1).astype(jnp.float32)
    h2 = jnp.maximum(dinv_ref[...] * accb + b2_ref[...], 0.0)
    rows = i * TBLK + lax.broadcasted_iota(jnp.int32, (TBLK, 1), 0)
    h2 = jnp.where(rows < N, h2, 0.0)
    sum_ref[...] += jnp.sum(h2, axis=0, keepdims=True)

    @pl.when(i == pl.num_programs(0) - 1)
    def _():
        mean = sum_ref[...] * (1.0 / N)
        out_ref[...] = (jnp.dot(mean, wc_ref[...],
                                preferred_element_type=jnp.float32)
                        + bc_ref[...])


def _tc3(acc2, dinv, b2, wc, bc):
    nb = N_PAD // TBLK
    return pl.pallas_call(
        _tc3_body,
        grid=(nb,),
        in_specs=[
            pl.BlockSpec((NC, TBLK, HALF), lambda i: (0, i, 0)),
            pl.BlockSpec((TBLK, 1), lambda i: (i, 0)),
            pl.BlockSpec((1, HID), lambda i: (0, 0)),
            pl.BlockSpec((HID, 1), lambda i: (0, 0)),
            pl.BlockSpec((1, 1), lambda i: (0, 0)),
        ],
        out_specs=pl.BlockSpec((1, 1), lambda i: (0, 0)),
        out_shape=jax.ShapeDtypeStruct((1, 1), jnp.float32),
        scratch_shapes=[pltpu.VMEM((1, HID), jnp.float32)],
    )(acc2, dinv, b2, wc, bc)


@jax.jit
def _run(x, edge_index, W_proj, b_proj, W1, b1, W2, b2, Wc, bc):
    xr = x.reshape(N, N_MOD)
    xp = jnp.zeros((N_PAD, N_MOD), jnp.float32).at[:N].set(xr)
    ei = jnp.full((2, E_PAD), DUMMY, jnp.int32).at[:, :E].set(edge_index)
    edges = ei.reshape(2, ROWS, 128).transpose(1, 0, 2)
    wf = W_proj @ W1
    bf = (b_proj @ W1)[None, :]

    degs = _deg(edges)
    g1, dinv = _tc1(xp, degs, wf, bf)
    acc1 = _conv(g1, edges)
    g2 = _tc2(acc1, dinv, W2, b1[None, :])
    acc2 = _conv(g2, edges)
    return _tc3(acc2, dinv, b2[None, :], Wc, bc[None, :])


def kernel(x, edge_index, W_proj, b_proj, W1, b1, W2, b2, Wc, bc):
    return _run(x, edge_index, W_proj, b_proj, W1, b1, W2, b2, Wc, bc)
